# baseline (speedup 1.0000x reference)
import functools

import jax
import jax.numpy as jnp
from jax import lax
from jax.experimental import pallas as pl
from jax.experimental.pallas import tpu as pltpu
from jax.experimental.pallas import tpu_sc as plsc

N_ROWS = 100000
N_FEAT = 128
N_SEG = 512
LANES = 16

N_FG = N_FEAT // LANES
N_RG = 4
ROWS_PER_RG = N_ROWS // N_RG
CHUNK = 2496
N_FULL = ROWS_PER_RG // CHUNK
TAIL_ROWS = ROWS_PER_RG - N_FULL * CHUNK

_mesh = plsc.VectorSubcoreMesh(core_axis_name="c", subcore_axis_name="s")
_params = pltpu.CompilerParams(use_tc_tiling_on_sc=False)

NEG_INF = float("-inf")


@functools.partial(
    pl.kernel,
    mesh=_mesh,
    compiler_params=_params,
    out_type=[
        jax.ShapeDtypeStruct((N_RG, N_SEG, N_FEAT), jnp.float32),
        jax.ShapeDtypeStruct((N_RG, N_SEG, N_FEAT), jnp.float32),
        jax.ShapeDtypeStruct((N_RG, N_SEG, LANES), jnp.float32),
    ],
    scratch_types=[
        pltpu.VMEM((2 * CHUNK, LANES), jnp.float32),
        pltpu.VMEM((2 * CHUNK + LANES,), jnp.int32),
        pltpu.VMEM((N_SEG, LANES), jnp.float32),
        pltpu.VMEM((N_SEG, LANES), jnp.float32),
        pltpu.VMEM((N_SEG, LANES), jnp.float32),
        pltpu.VMEM((2, LANES), jnp.float32),
        pltpu.SemaphoreType.DMA,
        pltpu.SemaphoreType.DMA,
        pltpu.SemaphoreType.DMA,
        pltpu.SemaphoreType.DMA,
    ],
)
def _phase_a(node_hbm, idx_hbm, sums_hbm, maxs_hbm, cnts_hbm,
             rowbuf, idxv, sumtab, maxtab, cnttab, accb,
             semr0, semr1, semi0, semi1):
    c = lax.axis_index("c")
    s = lax.axis_index("s")
    wid = s * 2 + c
    fg = wid % N_FG
    rg = wid // N_FG
    row0 = rg * ROWS_PER_RG
    col0 = fg * LANES

    zeros = jnp.zeros((LANES,), jnp.float32)
    ninf = jnp.full((LANES,), NEG_INF)

    def init_body(i, _):
        sumtab[i] = zeros
        maxtab[i] = ninf
        cnttab[i] = zeros
        return 0

    lax.fori_loop(0, N_SEG, init_body, 0)

    def row_body(i, carry):
        cur, cnt, sid = carry
        nsid = idxv[pl.ds(i + 1, LANES)][0]
        changed = sid != cur

        @pl.when(changed)
        def _flush():
            sumtab[cur] = accb[0]
            maxtab[cur] = accb[1]
            cnttab[cur] = jnp.full((LANES,), cnt)

        v = rowbuf[i]
        acc_s = jnp.where(changed, v, accb[0] + v)
        acc_m = jnp.where(changed, v, jnp.maximum(accb[1], v))
        cnt = jnp.where(changed, 1.0, cnt + 1.0)
        accb[0] = acc_s
        accb[1] = acc_m
        return sid, cnt, nsid

    BLK = 16

    _sems = ((semr0, semi0), (semr1, semi1))
    _chunks = [(i * CHUNK, CHUNK) for i in range(N_FULL)]
    if TAIL_ROWS:
        _chunks.append((N_FULL * CHUNK, TAIL_ROWS))

    def _row_cp(i):
        off, size = _chunks[i]
        half = i % 2
        return pltpu.make_async_copy(
            node_hbm.at[pl.ds(row0 + off, size), pl.ds(col0, LANES)],
            rowbuf.at[pl.ds(half * CHUNK, size)], _sems[half][0])

    def _idx_cp(i):
        off, size = _chunks[i]
        half = i % 2
        return pltpu.make_async_copy(
            idx_hbm.at[pl.ds(row0 + off, size)],
            idxv.at[pl.ds(half * CHUNK, size)], _sems[half][1])

    def blk_body_for(off):
      def blk_body(b, carry):
        ids_n = idxv[pl.ds(off + (b + 1) * BLK, LANES)]
        nfirst = ids_n[0]
        nlast = ids_n[LANES - 1]
        bbase = off + b * BLK
        cur, cnt, first, last = carry

        def fast(carry):
            cur, cnt = carry
            changed = first != cur
            vs = [rowbuf[bbase + j] for j in range(BLK)]
            ss = vs
            mm = vs
            while len(ss) > 1:
                ss = [ss[2 * j] + ss[2 * j + 1] for j in range(len(ss) // 2)]
                mm = [jnp.maximum(mm[2 * j], mm[2 * j + 1])
                      for j in range(len(mm) // 2)]
            bsum, bmax = ss[0], mm[0]

            @pl.when(changed)
            def _flush():
                sumtab[cur] = accb[0]
                maxtab[cur] = accb[1]
                cnttab[cur] = jnp.full((LANES,), cnt)

            acc_s = jnp.where(changed, bsum, accb[0] + bsum)
            acc_m = jnp.where(changed, bmax, jnp.maximum(accb[1], bmax))
            cnt = jnp.where(changed, float(BLK), cnt + float(BLK))
            accb[0] = acc_s
            accb[1] = acc_m
            return first, cnt

        def slow(carry):
            cur, cnt = carry
            cur, cnt, _ = lax.fori_loop(bbase, bbase + BLK, row_body,
                                        (cur, cnt, first))
            return cur, cnt

        cur, cnt = lax.cond(first == last, fast, slow, (cur, cnt))
        return cur, cnt, nfirst, nlast
      return blk_body

    accb[0] = zeros
    accb[1] = ninf
    _row_cp(0).start()
    _idx_cp(0).start()
    cur = jnp.int32(0)
    cnt = jnp.float32(0.0)
    for i in range(len(_chunks)):
        _, size = _chunks[i]
        boff = (i % 2) * CHUNK
        _row_cp(i).wait()
        _idx_cp(i).wait()
        if i + 1 < len(_chunks):
            _row_cp(i + 1).start()
            _idx_cp(i + 1).start()
        nb = size // BLK
        ids0 = idxv[pl.ds(boff, LANES)]
        nfirst = ids0[0]
        nlast = ids0[LANES - 1]
        if nb:
            carry4 = (cur, cnt, nfirst, nlast)
            cur, cnt, nfirst, nlast = lax.fori_loop(
                0, nb, blk_body_for(boff), carry4)
        if size % BLK:
            cur, cnt, _ = lax.fori_loop(boff + nb * BLK, boff + size,
                                        row_body, (cur, cnt, nfirst))

    sumtab[cur] = accb[0]
    maxtab[cur] = accb[1]
    cnttab[cur] = jnp.full((LANES,), cnt)

    pltpu.sync_copy(sumtab, sums_hbm.at[rg, :, pl.ds(col0, LANES)])
    pltpu.sync_copy(maxtab, maxs_hbm.at[rg, :, pl.ds(col0, LANES)])

    @pl.when(fg == 0)
    def _store_counts():
        pltpu.sync_copy(cnttab, cnts_hbm.at[rg])


N_WORKERS = 32
SEG_PER_W = N_SEG // N_WORKERS


@functools.partial(
    pl.kernel,
    mesh=_mesh,
    compiler_params=_params,
    out_type=jax.ShapeDtypeStruct((N_SEG * 2 * N_FG, LANES), jnp.float32),
    scratch_types=[
        pltpu.VMEM((N_RG * SEG_PER_W * N_FG, LANES), jnp.float32),
        pltpu.VMEM((N_RG * SEG_PER_W * N_FG, LANES), jnp.float32),
        pltpu.VMEM((N_RG * SEG_PER_W, LANES), jnp.float32),
        pltpu.VMEM((SEG_PER_W * 2 * N_FG, LANES), jnp.float32),
    ],
)
def _phase_b(sums_hbm, maxs_hbm, cnts_hbm, out_hbm, sbuf, mbuf, cbuf, obuf):
    c = lax.axis_index("c")
    s = lax.axis_index("s")
    wid = s * 2 + c
    seg0 = wid * SEG_PER_W
    W = SEG_PER_W * N_FG

    for rg in range(N_RG):
        pltpu.sync_copy(
            sums_hbm.at[pl.ds((rg * N_SEG + seg0) * N_FG, W)],
            sbuf.at[pl.ds(rg * W, W)])
        pltpu.sync_copy(
            maxs_hbm.at[pl.ds((rg * N_SEG + seg0) * N_FG, W)],
            mbuf.at[pl.ds(rg * W, W)])
        pltpu.sync_copy(
            cnts_hbm.at[pl.ds(rg * N_SEG + seg0, SEG_PER_W)],
            cbuf.at[pl.ds(rg * SEG_PER_W, SEG_PER_W)])

    def seg_body(k, _):
        cnt = (cbuf[k] + cbuf[SEG_PER_W + k]
               + cbuf[2 * SEG_PER_W + k] + cbuf[3 * SEG_PER_W + k])
        denom = jnp.maximum(cnt, 1.0)
        for f in range(N_FG):
            r = k * N_FG + f
            ssum = sbuf[r] + sbuf[W + r] + sbuf[2 * W + r] + sbuf[3 * W + r]
            obuf[k * 2 * N_FG + f] = ssum / denom
            mx = jnp.maximum(jnp.maximum(mbuf[r], mbuf[W + r]),
                             jnp.maximum(mbuf[2 * W + r], mbuf[3 * W + r]))
            obuf[k * 2 * N_FG + N_FG + f] = mx
        return 0

    lax.fori_loop(0, SEG_PER_W, seg_body, 0)
    pltpu.sync_copy(obuf, out_hbm.at[pl.ds(seg0 * 2 * N_FG,
                                           SEG_PER_W * 2 * N_FG)])


def kernel(node_repr, batch_idx):
    batch_idx = batch_idx.astype(jnp.int32)
    sums, maxs, cnts = _phase_a(node_repr, batch_idx)
    out = _phase_b(sums.reshape(N_RG * N_SEG * N_FG, LANES),
                   maxs.reshape(N_RG * N_SEG * N_FG, LANES),
                   cnts.reshape(N_RG * N_SEG, LANES))
    return out.reshape(N_SEG, 2 * N_FEAT)


# speedup vs baseline: 1.1476x; 1.1476x over previous
import functools

import jax
import jax.numpy as jnp
from jax import lax
from jax.experimental import pallas as pl
from jax.experimental.pallas import tpu as pltpu
from jax.experimental.pallas import tpu_sc as plsc

N_ROWS = 100000
N_FEAT = 128
N_SEG = 512
LANES = 16

N_FG = N_FEAT // LANES
N_RG = 4
ROWS_PER_RG = N_ROWS // N_RG
CHUNK = 2496
N_FULL = ROWS_PER_RG // CHUNK
TAIL_ROWS = ROWS_PER_RG - N_FULL * CHUNK

_mesh = plsc.VectorSubcoreMesh(core_axis_name="c", subcore_axis_name="s")
_params = pltpu.CompilerParams(use_tc_tiling_on_sc=False)

NEG_INF = float("-inf")


@functools.partial(
    pl.kernel,
    mesh=_mesh,
    compiler_params=_params,
    out_type=[
        jax.ShapeDtypeStruct((N_RG, N_SEG, N_FEAT), jnp.float32),
        jax.ShapeDtypeStruct((N_RG, N_SEG, N_FEAT), jnp.float32),
        jax.ShapeDtypeStruct((N_RG, N_SEG, LANES), jnp.float32),
    ],
    scratch_types=[
        pltpu.VMEM((2 * CHUNK, LANES), jnp.float32),
        pltpu.VMEM((2 * CHUNK + LANES,), jnp.int32),
        pltpu.VMEM((N_SEG, LANES), jnp.float32),
        pltpu.VMEM((N_SEG, LANES), jnp.float32),
        pltpu.VMEM((N_SEG, LANES), jnp.float32),
        pltpu.VMEM((2, LANES), jnp.float32),
        pltpu.SemaphoreType.DMA,
        pltpu.SemaphoreType.DMA,
        pltpu.SemaphoreType.DMA,
        pltpu.SemaphoreType.DMA,
    ],
)
def _phase_a(node_hbm, idx_hbm, sums_hbm, maxs_hbm, cnts_hbm,
             rowbuf, idxv, sumtab, maxtab, cnttab, accb,
             semr0, semr1, semi0, semi1):
    c = lax.axis_index("c")
    s = lax.axis_index("s")
    wid = s * 2 + c
    fg = wid % N_FG
    rg = wid // N_FG
    row0 = rg * ROWS_PER_RG
    col0 = fg * LANES

    zeros = jnp.zeros((LANES,), jnp.float32)
    ninf = jnp.full((LANES,), NEG_INF)

    def init_body(i, _):
        sumtab[i] = zeros
        maxtab[i] = ninf
        cnttab[i] = zeros
        return 0

    lax.fori_loop(0, N_SEG, init_body, 0)

    def row_body(i, carry):
        cur, cnt, sid = carry
        nsid = idxv[pl.ds(i + 1, LANES)][0]
        changed = sid != cur
        v = rowbuf[i]
        acc_s = jnp.where(changed, v, accb[0] + v)
        acc_m = jnp.where(changed, v, jnp.maximum(accb[1], v))
        cnt = jnp.where(changed, 1.0, cnt + 1.0)
        accb[0] = acc_s
        accb[1] = acc_m
        sumtab[sid] = acc_s
        maxtab[sid] = acc_m
        cnttab[sid] = jnp.full((LANES,), cnt)
        return sid, cnt, nsid

    BLK = 16

    _sems = ((semr0, semi0), (semr1, semi1))
    _chunks = [(i * CHUNK, CHUNK) for i in range(N_FULL)]
    if TAIL_ROWS:
        _chunks.append((N_FULL * CHUNK, TAIL_ROWS))

    def _row_cp(i):
        off, size = _chunks[i]
        half = i % 2
        return pltpu.make_async_copy(
            node_hbm.at[pl.ds(row0 + off, size), pl.ds(col0, LANES)],
            rowbuf.at[pl.ds(half * CHUNK, size)], _sems[half][0])

    def _idx_cp(i):
        off, size = _chunks[i]
        half = i % 2
        return pltpu.make_async_copy(
            idx_hbm.at[pl.ds(row0 + off, size)],
            idxv.at[pl.ds(half * CHUNK, size)], _sems[half][1])

    def _treereduce(bbase, n):
        vs = [rowbuf[bbase + j] for j in range(n)]
        ss = vs
        mm = vs
        while len(ss) > 1:
            ss = [ss[2 * j] + ss[2 * j + 1] for j in range(len(ss) // 2)]
            mm = [jnp.maximum(mm[2 * j], mm[2 * j + 1])
                  for j in range(len(mm) // 2)]
        return ss[0], mm[0]

    def _fast_for(bbase, first, n):
        def fast(carry):
            cur, cnt = carry
            changed = first != cur
            bsum, bmax = _treereduce(bbase, n)
            acc_s = jnp.where(changed, bsum, accb[0] + bsum)
            acc_m = jnp.where(changed, bmax, jnp.maximum(accb[1], bmax))
            cnt = jnp.where(changed, float(n), cnt + float(n))
            accb[0] = acc_s
            accb[1] = acc_m
            sumtab[first] = acc_s
            maxtab[first] = acc_m
            cnttab[first] = jnp.full((LANES,), cnt)
            return first, cnt
        return fast

    def _slow_for(bbase, first, n):
        def slow(carry):
            cur, cnt = carry
            cur, cnt, _ = lax.fori_loop(bbase, bbase + n, row_body,
                                        (cur, cnt, first))
            return cur, cnt
        return slow

    def pair_body_for(off):
      def pair_body(p, carry):
        nb = off + (p + 1) * 2 * BLK
        ids_a = idxv[pl.ds(nb, LANES)]
        ids_b = idxv[pl.ds(nb + BLK, LANES)]
        nf1 = ids_a[0]
        nl1 = ids_a[LANES - 1]
        nf2 = ids_b[0]
        nl2 = ids_b[LANES - 1]
        bbase = off + p * 2 * BLK
        cur, cnt, f1, l1, f2, l2 = carry

        def mixed(carry):
            carry = lax.cond(f1 == l1, _fast_for(bbase, f1, BLK),
                             _slow_for(bbase, f1, BLK), carry)
            return lax.cond(f2 == l2, _fast_for(bbase + BLK, f2, BLK),
                            _slow_for(bbase + BLK, f2, BLK), carry)

        cur, cnt = lax.cond(f1 == l2,
                            _fast_for(bbase, f1, 2 * BLK), mixed, (cur, cnt))
        return cur, cnt, nf1, nl1, nf2, nl2
      return pair_body

    accb[0] = zeros
    accb[1] = ninf
    _row_cp(0).start()
    _idx_cp(0).start()
    cur = jnp.int32(0)
    cnt = jnp.float32(0.0)
    for i in range(len(_chunks)):
        _, size = _chunks[i]
        boff = (i % 2) * CHUNK
        _row_cp(i).wait()
        _idx_cp(i).wait()
        if i + 1 < len(_chunks):
            _row_cp(i + 1).start()
            _idx_cp(i + 1).start()
        npair = size // (2 * BLK)
        ids_a = idxv[pl.ds(boff, LANES)]
        ids_b = idxv[pl.ds(boff + BLK, LANES)]
        nf1 = ids_a[0]
        nl1 = ids_a[LANES - 1]
        nf2 = ids_b[0]
        nl2 = ids_b[LANES - 1]
        if npair:
            carry6 = (cur, cnt, nf1, nl1, nf2, nl2)
            cur, cnt, nf1, nl1, nf2, nl2 = lax.fori_loop(
                0, npair, pair_body_for(boff), carry6)
        if size % (2 * BLK):
            cur, cnt, _ = lax.fori_loop(boff + npair * 2 * BLK, boff + size,
                                        row_body, (cur, cnt, nf1))

    pltpu.sync_copy(sumtab, sums_hbm.at[rg, :, pl.ds(col0, LANES)])
    pltpu.sync_copy(maxtab, maxs_hbm.at[rg, :, pl.ds(col0, LANES)])

    @pl.when(fg == 0)
    def _store_counts():
        pltpu.sync_copy(cnttab, cnts_hbm.at[rg])


N_WORKERS = 32
SEG_PER_W = N_SEG // N_WORKERS


@functools.partial(
    pl.kernel,
    mesh=_mesh,
    compiler_params=_params,
    out_type=jax.ShapeDtypeStruct((N_SEG * 2 * N_FG, LANES), jnp.float32),
    scratch_types=[
        pltpu.VMEM((N_RG * SEG_PER_W * N_FG, LANES), jnp.float32),
        pltpu.VMEM((N_RG * SEG_PER_W * N_FG, LANES), jnp.float32),
        pltpu.VMEM((N_RG * SEG_PER_W, LANES), jnp.float32),
        pltpu.VMEM((SEG_PER_W * 2 * N_FG, LANES), jnp.float32),
    ],
)
def _phase_b(sums_hbm, maxs_hbm, cnts_hbm, out_hbm, sbuf, mbuf, cbuf, obuf):
    c = lax.axis_index("c")
    s = lax.axis_index("s")
    wid = s * 2 + c
    seg0 = wid * SEG_PER_W
    W = SEG_PER_W * N_FG

    for rg in range(N_RG):
        pltpu.sync_copy(
            sums_hbm.at[pl.ds((rg * N_SEG + seg0) * N_FG, W)],
            sbuf.at[pl.ds(rg * W, W)])
        pltpu.sync_copy(
            maxs_hbm.at[pl.ds((rg * N_SEG + seg0) * N_FG, W)],
            mbuf.at[pl.ds(rg * W, W)])
        pltpu.sync_copy(
            cnts_hbm.at[pl.ds(rg * N_SEG + seg0, SEG_PER_W)],
            cbuf.at[pl.ds(rg * SEG_PER_W, SEG_PER_W)])

    def seg_body(k, _):
        cnt = (cbuf[k] + cbuf[SEG_PER_W + k]
               + cbuf[2 * SEG_PER_W + k] + cbuf[3 * SEG_PER_W + k])
        denom = jnp.maximum(cnt, 1.0)
        for f in range(N_FG):
            r = k * N_FG + f
            ssum = sbuf[r] + sbuf[W + r] + sbuf[2 * W + r] + sbuf[3 * W + r]
            obuf[k * 2 * N_FG + f] = ssum / denom
            mx = jnp.maximum(jnp.maximum(mbuf[r], mbuf[W + r]),
                             jnp.maximum(mbuf[2 * W + r], mbuf[3 * W + r]))
            obuf[k * 2 * N_FG + N_FG + f] = mx
        return 0

    lax.fori_loop(0, SEG_PER_W, seg_body, 0)
    pltpu.sync_copy(obuf, out_hbm.at[pl.ds(seg0 * 2 * N_FG,
                                           SEG_PER_W * 2 * N_FG)])


def kernel(node_repr, batch_idx):
    batch_idx = batch_idx.astype(jnp.int32)
    sums, maxs, cnts = _phase_a(node_repr, batch_idx)
    out = _phase_b(sums.reshape(N_RG * N_SEG * N_FG, LANES),
                   maxs.reshape(N_RG * N_SEG * N_FG, LANES),
                   cnts.reshape(N_RG * N_SEG, LANES))
    return out.reshape(N_SEG, 2 * N_FEAT)
